# Initial kernel scaffold; baseline (speedup 1.0000x reference)
import jax
import jax.numpy as jnp
from jax.experimental import pallas as pl


def kernel(state, program, step_emb, lib_W, lib_b):
    raise NotImplementedError("write your pallas kernel here")


# speedup vs baseline: 9.0358x; 9.0358x over previous
import jax
import jax.numpy as jnp
from jax.experimental import pallas as pl

_BLK = 1024


def _exec_kernel(prog_ref, emb_ref, libw_ref, libb_ref, state_ref, out_ref):
    prog = prog_ref[...]
    p = jax.nn.softmax(prog, axis=-1)
    w = jnp.dot(p, libw_ref[...], preferred_element_type=jnp.float32)
    b = jnp.dot(p, libb_ref[...], preferred_element_type=jnp.float32)
    c = emb_ref[...] * w + b
    x = state_ref[...]
    for t in range(prog.shape[0]):
        x = jnp.tanh(x * w[t][None, :] + c[t][None, :])
    out_ref[...] = x


def kernel(state, program, step_emb, lib_W, lib_b):
    batch, d = state.shape
    s, prims = program.shape
    final = pl.pallas_call(
        _exec_kernel,
        grid=(batch // _BLK,),
        in_specs=[
            pl.BlockSpec((s, prims), lambda i: (0, 0)),
            pl.BlockSpec((s, d), lambda i: (0, 0)),
            pl.BlockSpec((prims, d), lambda i: (0, 0)),
            pl.BlockSpec((prims, d), lambda i: (0, 0)),
            pl.BlockSpec((_BLK, d), lambda i: (i, 0)),
        ],
        out_specs=pl.BlockSpec((_BLK, d), lambda i: (i, 0)),
        out_shape=jax.ShapeDtypeStruct((batch, d), jnp.float32),
    )(program, step_emb, lib_W, lib_b, state)
    return (final, program)


# speedup vs baseline: 9.1103x; 1.0082x over previous
import jax
import jax.numpy as jnp
from jax.experimental import pallas as pl
from jax.experimental.pallas import tpu as pltpu

_BLK = 1024


def _prep_kernel(prog_ref, emb_ref, libw_ref, libb_ref, w_ref, c_ref):
    p = jax.nn.softmax(prog_ref[...], axis=-1)
    w = jnp.dot(p, libw_ref[...], preferred_element_type=jnp.float32)
    b = jnp.dot(p, libb_ref[...], preferred_element_type=jnp.float32)
    w_ref[...] = w
    c_ref[...] = emb_ref[...] * w + b


def _exec_kernel(w_ref, c_ref, state_ref, out_ref):
    w = w_ref[...]
    c = c_ref[...]
    x = state_ref[...]
    for t in range(w.shape[0]):
        x = jnp.tanh(x * w[t][None, :] + c[t][None, :])
    out_ref[...] = x


def kernel(state, program, step_emb, lib_W, lib_b):
    batch, d = state.shape
    s, prims = program.shape
    w, c = pl.pallas_call(
        _prep_kernel,
        out_shape=(
            jax.ShapeDtypeStruct((s, d), jnp.float32),
            jax.ShapeDtypeStruct((s, d), jnp.float32),
        ),
    )(program, step_emb, lib_W, lib_b)
    final = pl.pallas_call(
        _exec_kernel,
        grid=(batch // _BLK,),
        in_specs=[
            pl.BlockSpec((s, d), lambda i: (0, 0)),
            pl.BlockSpec((s, d), lambda i: (0, 0)),
            pl.BlockSpec((_BLK, d), lambda i: (i, 0)),
        ],
        out_specs=pl.BlockSpec((_BLK, d), lambda i: (i, 0)),
        out_shape=jax.ShapeDtypeStruct((batch, d), jnp.float32),
        compiler_params=pltpu.CompilerParams(
            dimension_semantics=("parallel",),
        ),
    )(w, c, state)
    return (final, program)


# speedup vs baseline: 9.2275x; 1.0129x over previous
import jax
import jax.numpy as jnp
from jax.experimental import pallas as pl
from jax.experimental.pallas import tpu as pltpu

_BLK = 4096


def _prep_kernel(prog_ref, emb_ref, libw_ref, libb_ref, w_ref, c_ref):
    p = jax.nn.softmax(prog_ref[...], axis=-1)
    w = jnp.dot(p, libw_ref[...], preferred_element_type=jnp.float32)
    b = jnp.dot(p, libb_ref[...], preferred_element_type=jnp.float32)
    w_ref[...] = w
    c_ref[...] = emb_ref[...] * w + b


def _exec_kernel(w_ref, c_ref, state_ref, out_ref):
    w = w_ref[...]
    c = c_ref[...]
    x = state_ref[...]
    for t in range(w.shape[0]):
        x = jnp.tanh(x * w[t][None, :] + c[t][None, :])
    out_ref[...] = x


def kernel(state, program, step_emb, lib_W, lib_b):
    batch, d = state.shape
    s, prims = program.shape
    w, c = pl.pallas_call(
        _prep_kernel,
        out_shape=(
            jax.ShapeDtypeStruct((s, d), jnp.float32),
            jax.ShapeDtypeStruct((s, d), jnp.float32),
        ),
    )(program, step_emb, lib_W, lib_b)
    final = pl.pallas_call(
        _exec_kernel,
        grid=(batch // _BLK,),
        in_specs=[
            pl.BlockSpec((s, d), lambda i: (0, 0)),
            pl.BlockSpec((s, d), lambda i: (0, 0)),
            pl.BlockSpec((_BLK, d), lambda i: (i, 0)),
        ],
        out_specs=pl.BlockSpec((_BLK, d), lambda i: (i, 0)),
        out_shape=jax.ShapeDtypeStruct((batch, d), jnp.float32),
        compiler_params=pltpu.CompilerParams(
            dimension_semantics=("parallel",),
        ),
    )(w, c, state)
    return (final, program)
